# baseline (speedup 1.0000x reference)
import jax
import jax.numpy as jnp
from jax import lax
from jax.experimental import pallas as pl
from jax.experimental.pallas import tpu as pltpu
from jax.experimental.pallas import tpu_sc as plsc

V = 1000000
D = 64
B = 16384
C = 20
NEG = 20
TN = NEG + 1
W = D // 2

NUM_WORKERS = 32
PER_W = B // NUM_WORKERS
E = 16
CHUNKS = PER_W // E
L = 16

PACK_RB = 200
PACK_NBLK = V // PACK_RB


def _pack_body(src_hbm, dst_hbm, x0, x1, w0, w1, si0, si1, so0, so1):
  wid = lax.axis_index("s") * 2 + lax.axis_index("c")
  base = PACK_NBLK // NUM_WORKERS
  cnt = base + jnp.where(wid < PACK_NBLK % NUM_WORKERS, 1, 0)
  start = wid * base + jnp.minimum(wid, PACK_NBLK % NUM_WORKERS)
  sets = ((x0, w0, si0, so0), (x1, w1, si1, so1))
  XB = PACK_RB * D
  WB = PACK_RB * W

  def fire_in(i, s):
    x, _, si, _ = s
    pltpu.async_copy(src_hbm.at[pl.ds((start + i) * XB, XB)], x, si)

  def compute(s):
    x, w, _, _ = s

    @pl.loop(0, PACK_RB, unroll=8)
    def _rows(r):
      for q in range(2):
        a = x[pl.ds(r * D + q * L, L)]
        bb = x[pl.ds(r * D + W + q * L, L)]
        wv = plsc.bitcast(
            plsc.pack(a, bb, format=plsc.PackFormat.INTERLEAVED),
            jnp.int32)
        w[pl.ds(r * W + q * L, L)] = wv

  def proc(i, s):
    x, w, si, so = s
    pltpu.make_async_copy(src_hbm.at[pl.ds(0, XB)], x, si).wait()

    @pl.when(i >= 2)
    def _dr():
      pltpu.make_async_copy(w, dst_hbm.at[pl.ds(0, WB)], so).wait()

    compute(s)
    pltpu.async_copy(w, dst_hbm.at[pl.ds((start + i) * WB, WB)], so)

  fire_in(0, sets[0])
  fire_in(1, sets[1])

  @pl.loop(0, cnt // 2)
  def _blocks(t):
    i = t * 2
    proc(i, sets[0])

    @pl.when(i + 2 < cnt)
    def _f0():
      fire_in(i + 2, sets[0])

    proc(i + 1, sets[1])

    @pl.when(i + 3 < cnt)
    def _f1():
      fire_in(i + 3, sets[1])

  @pl.when(cnt % 2 == 1)
  def _tail():
    proc(cnt - 1, sets[0])

  for s in sets:
    _, w, _, so = s
    pltpu.make_async_copy(w, dst_hbm.at[pl.ds(0, WB)], so).wait()


def _pack_table(table):
  mesh = plsc.VectorSubcoreMesh(core_axis_name="c", subcore_axis_name="s")
  f = pl.kernel(
      _pack_body,
      out_type=jax.ShapeDtypeStruct((V * W,), jnp.int32),
      mesh=mesh,
      scratch_types=[
          pltpu.VMEM((PACK_RB * D,), jnp.float32),
          pltpu.VMEM((PACK_RB * D,), jnp.float32),
          pltpu.VMEM((PACK_RB * W,), jnp.int32),
          pltpu.VMEM((PACK_RB * W,), jnp.int32),
          pltpu.SemaphoreType.DMA,
          pltpu.SemaphoreType.DMA,
          pltpu.SemaphoreType.DMA,
          pltpu.SemaphoreType.DMA,
      ],
      compiler_params=pltpu.CompilerParams(
          needs_layout_passes=False, use_tc_tiling_on_sc=False),
  )
  return f(table.reshape(-1))


def _body(ctx_idx_hbm, tn_idx_hbm, iw_hbm, ow_hbm, out_hbm,
          ctx_idx_v, tn_idx_v, crows0, trows0, crows1, trows1,
          scores_v, sem0, sem1):
  wid = lax.axis_index("s") * 2 + lax.axis_index("c")
  wbase = wid * PER_W
  ev = lax.iota(jnp.int32, L)
  inv_c = jnp.float32(1.0 / C)
  evC = ev * C
  evTN = ev * TN
  himask = jnp.full((L,), -65536, jnp.int32)

  pltpu.sync_copy(ctx_idx_hbm.at[pl.ds(wbase * C, PER_W * C)], ctx_idx_v)
  pltpu.sync_copy(tn_idx_hbm.at[pl.ds(wbase * TN, PER_W * TN)], tn_idx_v)

  bufs = ((crows0, trows0, sem0), (crows1, trows1, sem1))

  def fire(c, crows, trows, sem):
    pltpu.async_copy(
        iw_hbm.at[ctx_idx_v.at[pl.ds(c * (E * C), E * C)]], crows, sem)
    pltpu.async_copy(
        ow_hbm.at[tn_idx_v.at[pl.ds(c * (E * TN), E * TN)]], trows, sem)

  def drain(crows, trows, sem):
    pltpu.make_async_copy(
        iw_hbm.at[ctx_idx_v.at[pl.ds(0, E * C)]], crows, sem).wait()
    pltpu.make_async_copy(
        ow_hbm.at[tn_idx_v.at[pl.ds(0, E * TN)]], trows, sem).wait()

  def unpack(word):
    f_lo = plsc.bitcast(word << 16, jnp.float32)
    f_hi = plsc.bitcast(word & himask, jnp.float32)
    return f_lo, f_hi

  def compute(c, crows, trows):
    cev = c * E + ev

    def word_scores(k):
      kcol = jnp.full((L,), k, jnp.int32)
      s_lo, s_hi = unpack(plsc.load_gather(crows, [evC, kcol]))
      for r in range(1, C):
        lo, hi = unpack(plsc.load_gather(crows, [evC + r, kcol]))
        s_lo = s_lo + lo
        s_hi = s_hi + hi
      a_lo = s_lo * inv_c
      a_hi = s_hi * inv_c
      out = []
      for j in range(TN):
        lo, hi = unpack(plsc.load_gather(trows, [evTN + j, kcol]))
        out.append(a_lo * lo + a_hi * hi)
      return out

    sc0 = word_scores(0)
    for j in range(TN):
      plsc.store_scatter(
          scores_v, [cev, jnp.full((L,), j, jnp.int32)], sc0[j])

    @pl.loop(1, W)
    def _words(k):
      sck = word_scores(k)
      for j in range(TN):
        plsc.addupdate_scatter(
            scores_v, [cev, jnp.full((L,), j, jnp.int32)], sck[j])

  fire(0, *bufs[0])

  @pl.loop(0, CHUNKS, step=2)
  def _chunks(c):
    fire(c + 1, *bufs[1])
    drain(*bufs[0])
    compute(c, bufs[0][0], bufs[0][1])

    @pl.when(c + 2 < CHUNKS)
    def _prefetch():
      fire(c + 2, *bufs[0])

    drain(*bufs[1])
    compute(c + 1, bufs[1][0], bufs[1][1])

  pltpu.sync_copy(scores_v, out_hbm.at[pl.ds(wbase, PER_W)])


def _cbow_scores(ctx_idx, tn_idx, iw_packed, ow_packed):
  mesh = plsc.VectorSubcoreMesh(core_axis_name="c", subcore_axis_name="s")
  f = pl.kernel(
      _body,
      out_type=jax.ShapeDtypeStruct((B, TN), jnp.float32),
      mesh=mesh,
      scratch_types=[
          pltpu.VMEM((PER_W * C,), jnp.int32),
          pltpu.VMEM((PER_W * TN,), jnp.int32),
          pltpu.VMEM((E * C, W), jnp.int32),
          pltpu.VMEM((E * TN, W), jnp.int32),
          pltpu.VMEM((E * C, W), jnp.int32),
          pltpu.VMEM((E * TN, W), jnp.int32),
          pltpu.VMEM((PER_W, TN), jnp.float32),
          pltpu.SemaphoreType.DMA,
          pltpu.SemaphoreType.DMA,
      ],
      compiler_params=pltpu.CompilerParams(
          needs_layout_passes=False, use_tc_tiling_on_sc=False),
  )
  return f(ctx_idx, tn_idx, iw_packed, ow_packed)


def kernel(context_words, target_word, negative_samples, input_weight,
           output_weight):
  ctx_idx = context_words.astype(jnp.int32).reshape(-1)
  tn_idx = jnp.concatenate(
      [target_word.astype(jnp.int32)[:, None],
       negative_samples.astype(jnp.int32)], axis=1).reshape(-1)
  iw_packed = _pack_table(input_weight).reshape(V, W)
  ow_packed = _pack_table(output_weight).reshape(V, W)
  scores = _cbow_scores(ctx_idx, tn_idx, iw_packed, ow_packed)
  return scores[:, 0], scores[:, 1:]


# speedup vs baseline: 1.0859x; 1.0859x over previous
import jax
import jax.numpy as jnp
from jax import lax
from jax.experimental import pallas as pl
from jax.experimental.pallas import tpu as pltpu
from jax.experimental.pallas import tpu_sc as plsc

V = 1000000
D = 64
B = 16384
C = 20
NEG = 20
TN = NEG + 1
W = D // 2

NUM_WORKERS = 32
PER_W = B // NUM_WORKERS
E = 16
CHUNKS = PER_W // E
L = 16

PACK_RB = 200
PACK_NBLK = V // PACK_RB


def _pack_body(iw_hbm, ow_hbm, o1_hbm, o2_hbm,
               x0, x1, w0, w1, si0, si1, so0, so1):
  wid = lax.axis_index("s") * 2 + lax.axis_index("c")
  base = PACK_NBLK // NUM_WORKERS
  cnt = base + jnp.where(wid < PACK_NBLK % NUM_WORKERS, 1, 0)
  start = wid * base + jnp.minimum(wid, PACK_NBLK % NUM_WORKERS)
  sets = ((x0, w0, si0, so0), (x1, w1, si1, so1))

  for src, dst in ((iw_hbm, o1_hbm), (ow_hbm, o2_hbm)):

    def fire_in(i, s):
      x, _, si, _ = s
      pltpu.async_copy(
          src.at[pl.ds((start + i) * PACK_RB, PACK_RB), :], x, si)

    def compute(s):
      x, w, _, _ = s

      @pl.loop(0, PACK_RB, unroll=8)
      def _rows(r):
        for q in range(2):
          a = x[r, pl.ds(q * L, L)]
          bb = x[r, pl.ds(W + q * L, L)]
          wv = plsc.bitcast(
              plsc.pack(a, bb, format=plsc.PackFormat.INTERLEAVED),
              jnp.int32)
          w[pl.ds(r * W + q * L, L)] = wv

    def proc(i, s):
      x, w, si, so = s
      pltpu.make_async_copy(
          src.at[pl.ds(0, PACK_RB), :], x, si).wait()

      @pl.when(i >= 2)
      def _dr():
        pltpu.make_async_copy(w, dst.at[pl.ds(0, PACK_RB * W)], so).wait()

      compute(s)
      pltpu.async_copy(w, dst.at[pl.ds((start + i) * PACK_RB * W,
                                       PACK_RB * W)], so)

    fire_in(0, sets[0])
    fire_in(1, sets[1])

    @pl.loop(0, cnt // 2)
    def _blocks(t):
      i = t * 2
      proc(i, sets[0])

      @pl.when(i + 2 < cnt)
      def _f0():
        fire_in(i + 2, sets[0])

      proc(i + 1, sets[1])

      @pl.when(i + 3 < cnt)
      def _f1():
        fire_in(i + 3, sets[1])

    @pl.when(cnt % 2 == 1)
    def _tail():
      proc(cnt - 1, sets[0])

    for s in sets:
      _, w, _, so = s
      pltpu.make_async_copy(w, dst.at[pl.ds(0, PACK_RB * W)], so).wait()


def _pack_tables(iw, ow):
  mesh = plsc.VectorSubcoreMesh(core_axis_name="c", subcore_axis_name="s")
  f = pl.kernel(
      _pack_body,
      out_type=(jax.ShapeDtypeStruct((V * W,), jnp.int32),
                jax.ShapeDtypeStruct((V * W,), jnp.int32)),
      mesh=mesh,
      scratch_types=[
          pltpu.VMEM((PACK_RB, D), jnp.float32),
          pltpu.VMEM((PACK_RB, D), jnp.float32),
          pltpu.VMEM((PACK_RB * W,), jnp.int32),
          pltpu.VMEM((PACK_RB * W,), jnp.int32),
          pltpu.SemaphoreType.DMA,
          pltpu.SemaphoreType.DMA,
          pltpu.SemaphoreType.DMA,
          pltpu.SemaphoreType.DMA,
      ],
      compiler_params=pltpu.CompilerParams(
          needs_layout_passes=False, use_tc_tiling_on_sc=True),
  )
  return f(iw, ow)


def _body(ctx_idx_hbm, tn_idx_hbm, iw_hbm, ow_hbm, out_hbm,
          ctx_idx_v, tn_idx_v, crows0, trows0, crows1, trows1,
          scores_v, sem0, sem1):
  wid = lax.axis_index("s") * 2 + lax.axis_index("c")
  wbase = wid * PER_W
  ev = lax.iota(jnp.int32, L)
  inv_c = jnp.float32(1.0 / C)
  evC = ev * C
  evTN = ev * TN
  himask = jnp.full((L,), -65536, jnp.int32)

  pltpu.sync_copy(ctx_idx_hbm.at[pl.ds(wbase * C, PER_W * C)], ctx_idx_v)
  pltpu.sync_copy(tn_idx_hbm.at[pl.ds(wbase * TN, PER_W * TN)], tn_idx_v)

  bufs = ((crows0, trows0, sem0), (crows1, trows1, sem1))

  def fire(c, crows, trows, sem):
    pltpu.async_copy(
        iw_hbm.at[ctx_idx_v.at[pl.ds(c * (E * C), E * C)]], crows, sem)
    pltpu.async_copy(
        ow_hbm.at[tn_idx_v.at[pl.ds(c * (E * TN), E * TN)]], trows, sem)

  def drain(crows, trows, sem):
    pltpu.make_async_copy(
        iw_hbm.at[ctx_idx_v.at[pl.ds(0, E * C)]], crows, sem).wait()
    pltpu.make_async_copy(
        ow_hbm.at[tn_idx_v.at[pl.ds(0, E * TN)]], trows, sem).wait()

  def unpack(word):
    f_lo = plsc.bitcast(word << 16, jnp.float32)
    f_hi = plsc.bitcast(word & himask, jnp.float32)
    return f_lo, f_hi

  def compute(c, crows, trows):
    cev = c * E + ev

    def word_scores(k):
      kcol = jnp.full((L,), k, jnp.int32)
      s_lo, s_hi = unpack(plsc.load_gather(crows, [evC, kcol]))
      for r in range(1, C):
        lo, hi = unpack(plsc.load_gather(crows, [evC + r, kcol]))
        s_lo = s_lo + lo
        s_hi = s_hi + hi
      a_lo = s_lo * inv_c
      a_hi = s_hi * inv_c
      out = []
      for j in range(TN):
        lo, hi = unpack(plsc.load_gather(trows, [evTN + j, kcol]))
        out.append(a_lo * lo + a_hi * hi)
      return out

    sc0 = word_scores(0)
    for j in range(TN):
      plsc.store_scatter(
          scores_v, [cev, jnp.full((L,), j, jnp.int32)], sc0[j])

    @pl.loop(1, W)
    def _words(k):
      sck = word_scores(k)
      for j in range(TN):
        plsc.addupdate_scatter(
            scores_v, [cev, jnp.full((L,), j, jnp.int32)], sck[j])

  fire(0, *bufs[0])

  @pl.loop(0, CHUNKS, step=2)
  def _chunks(c):
    fire(c + 1, *bufs[1])
    drain(*bufs[0])
    compute(c, bufs[0][0], bufs[0][1])

    @pl.when(c + 2 < CHUNKS)
    def _prefetch():
      fire(c + 2, *bufs[0])

    drain(*bufs[1])
    compute(c + 1, bufs[1][0], bufs[1][1])

  pltpu.sync_copy(scores_v, out_hbm.at[pl.ds(wbase, PER_W)])


def _cbow_scores(ctx_idx, tn_idx, iw_packed, ow_packed):
  mesh = plsc.VectorSubcoreMesh(core_axis_name="c", subcore_axis_name="s")
  f = pl.kernel(
      _body,
      out_type=jax.ShapeDtypeStruct((B, TN), jnp.float32),
      mesh=mesh,
      scratch_types=[
          pltpu.VMEM((PER_W * C,), jnp.int32),
          pltpu.VMEM((PER_W * TN,), jnp.int32),
          pltpu.VMEM((E * C, W), jnp.int32),
          pltpu.VMEM((E * TN, W), jnp.int32),
          pltpu.VMEM((E * C, W), jnp.int32),
          pltpu.VMEM((E * TN, W), jnp.int32),
          pltpu.VMEM((PER_W, TN), jnp.float32),
          pltpu.SemaphoreType.DMA,
          pltpu.SemaphoreType.DMA,
      ],
      compiler_params=pltpu.CompilerParams(
          needs_layout_passes=False, use_tc_tiling_on_sc=False),
  )
  return f(ctx_idx, tn_idx, iw_packed, ow_packed)


def kernel(context_words, target_word, negative_samples, input_weight,
           output_weight):
  ctx_idx = context_words.astype(jnp.int32).reshape(-1)
  tn_idx = jnp.concatenate(
      [target_word.astype(jnp.int32)[:, None],
       negative_samples.astype(jnp.int32)], axis=1).reshape(-1)
  p1, p2 = _pack_tables(input_weight, output_weight)
  iw_packed = p1.reshape(V, W)
  ow_packed = p2.reshape(V, W)
  scores = _cbow_scores(ctx_idx, tn_idx, iw_packed, ow_packed)
  return scores[:, 0], scores[:, 1:]


# speedup vs baseline: 1.2022x; 1.1071x over previous
import jax
import jax.numpy as jnp
from jax import lax
from jax.experimental import pallas as pl
from jax.experimental.pallas import tpu as pltpu
from jax.experimental.pallas import tpu_sc as plsc

V = 1000000
D = 64
B = 16384
C = 20
NEG = 20
TN = NEG + 1
W = D // 2

NUM_WORKERS = 32
PER_W = B // NUM_WORKERS
E = 16
CHUNKS = PER_W // E
L = 16

PACK_RB = 200
PACK_NBLK = V // PACK_RB


def _pack_body(src, dst,
               x0, x1, w0, w1, si0, si1, so0, so1):
  wid = lax.axis_index("s") * 2 + lax.axis_index("c")
  base = PACK_NBLK // NUM_WORKERS
  cnt = base + jnp.where(wid < PACK_NBLK % NUM_WORKERS, 1, 0)
  start = wid * base + jnp.minimum(wid, PACK_NBLK % NUM_WORKERS)
  sets = ((x0, w0, si0, so0), (x1, w1, si1, so1))

  if True:

    def fire_in(i, s):
      x, _, si, _ = s
      pltpu.async_copy(
          src.at[pl.ds((start + i) * PACK_RB, PACK_RB), :], x, si)

    def compute(s):
      x, w, _, _ = s

      @pl.loop(0, PACK_RB, unroll=8)
      def _rows(r):
        for q in range(2):
          a = x[r, pl.ds(q * L, L)]
          bb = x[r, pl.ds(W + q * L, L)]
          wv = plsc.bitcast(
              plsc.pack(a, bb, format=plsc.PackFormat.INTERLEAVED),
              jnp.int32)
          w[pl.ds(r * W + q * L, L)] = wv

    def proc(i, s):
      x, w, si, so = s
      pltpu.make_async_copy(
          src.at[pl.ds(0, PACK_RB), :], x, si).wait()

      @pl.when(i >= 2)
      def _dr():
        pltpu.make_async_copy(w, dst.at[pl.ds(0, PACK_RB * W)], so).wait()

      compute(s)
      pltpu.async_copy(w, dst.at[pl.ds((start + i) * PACK_RB * W,
                                       PACK_RB * W)], so)

    fire_in(0, sets[0])
    fire_in(1, sets[1])

    @pl.loop(0, cnt // 2)
    def _blocks(t):
      i = t * 2
      proc(i, sets[0])

      @pl.when(i + 2 < cnt)
      def _f0():
        fire_in(i + 2, sets[0])

      proc(i + 1, sets[1])

      @pl.when(i + 3 < cnt)
      def _f1():
        fire_in(i + 3, sets[1])

    @pl.when(cnt % 2 == 1)
    def _tail():
      proc(cnt - 1, sets[0])

    for s in sets:
      _, w, _, so = s
      pltpu.make_async_copy(w, dst.at[pl.ds(0, PACK_RB * W)], so).wait()


def _pack_table(table):
  mesh = plsc.VectorSubcoreMesh(core_axis_name="c", subcore_axis_name="s")
  f = pl.kernel(
      _pack_body,
      out_type=jax.ShapeDtypeStruct((V * W,), jnp.int32),
      mesh=mesh,
      scratch_types=[
          pltpu.VMEM((PACK_RB, D), jnp.float32),
          pltpu.VMEM((PACK_RB, D), jnp.float32),
          pltpu.VMEM((PACK_RB * W,), jnp.int32),
          pltpu.VMEM((PACK_RB * W,), jnp.int32),
          pltpu.SemaphoreType.DMA,
          pltpu.SemaphoreType.DMA,
          pltpu.SemaphoreType.DMA,
          pltpu.SemaphoreType.DMA,
      ],
      compiler_params=pltpu.CompilerParams(
          needs_layout_passes=False, use_tc_tiling_on_sc=True),
  )
  return f(table)


def _body(ctx_idx_hbm, tn_idx_hbm, iw_hbm, ow_hbm, out_hbm,
          ctx_idx_v, tn_idx_v, crows0, trows0, crows1, trows1,
          scores_v, sem0, sem1):
  wid = lax.axis_index("s") * 2 + lax.axis_index("c")
  wbase = wid * PER_W
  ev = lax.iota(jnp.int32, L)
  inv_c = jnp.float32(1.0 / C)
  evC = ev * C
  evTN = ev * TN
  himask = jnp.full((L,), -65536, jnp.int32)

  pltpu.sync_copy(ctx_idx_hbm.at[pl.ds(wbase * C, PER_W * C)], ctx_idx_v)
  pltpu.sync_copy(tn_idx_hbm.at[pl.ds(wbase * TN, PER_W * TN)], tn_idx_v)

  bufs = ((crows0, trows0, sem0), (crows1, trows1, sem1))

  def fire(c, crows, trows, sem):
    pltpu.async_copy(
        iw_hbm.at[ctx_idx_v.at[pl.ds(c * (E * C), E * C)]], crows, sem)
    pltpu.async_copy(
        ow_hbm.at[tn_idx_v.at[pl.ds(c * (E * TN), E * TN)]], trows, sem)

  def drain(crows, trows, sem):
    pltpu.make_async_copy(
        iw_hbm.at[ctx_idx_v.at[pl.ds(0, E * C)]], crows, sem).wait()
    pltpu.make_async_copy(
        ow_hbm.at[tn_idx_v.at[pl.ds(0, E * TN)]], trows, sem).wait()

  def unpack(word):
    f_lo = plsc.bitcast(word << 16, jnp.float32)
    f_hi = plsc.bitcast(word & himask, jnp.float32)
    return f_lo, f_hi

  def compute(c, crows, trows):
    cev = c * E + ev

    def word_scores(k):
      kcol = jnp.full((L,), k, jnp.int32)
      s_lo, s_hi = unpack(plsc.load_gather(crows, [evC, kcol]))
      for r in range(1, C):
        lo, hi = unpack(plsc.load_gather(crows, [evC + r, kcol]))
        s_lo = s_lo + lo
        s_hi = s_hi + hi
      a_lo = s_lo * inv_c
      a_hi = s_hi * inv_c
      out = []
      for j in range(TN):
        lo, hi = unpack(plsc.load_gather(trows, [evTN + j, kcol]))
        out.append(a_lo * lo + a_hi * hi)
      return out

    sc0 = word_scores(0)
    for j in range(TN):
      plsc.store_scatter(
          scores_v, [cev, jnp.full((L,), j, jnp.int32)], sc0[j])

    @pl.loop(1, W)
    def _words(k):
      sck = word_scores(k)
      for j in range(TN):
        plsc.addupdate_scatter(
            scores_v, [cev, jnp.full((L,), j, jnp.int32)], sck[j])

  fire(0, *bufs[0])

  @pl.loop(0, CHUNKS, step=2)
  def _chunks(c):
    fire(c + 1, *bufs[1])
    drain(*bufs[0])
    compute(c, bufs[0][0], bufs[0][1])

    @pl.when(c + 2 < CHUNKS)
    def _prefetch():
      fire(c + 2, *bufs[0])

    drain(*bufs[1])
    compute(c + 1, bufs[1][0], bufs[1][1])

  pltpu.sync_copy(scores_v, out_hbm.at[pl.ds(wbase, PER_W)])


def _cbow_scores(ctx_idx, tn_idx, iw_packed, ow_packed):
  mesh = plsc.VectorSubcoreMesh(core_axis_name="c", subcore_axis_name="s")
  f = pl.kernel(
      _body,
      out_type=jax.ShapeDtypeStruct((B, TN), jnp.float32),
      mesh=mesh,
      scratch_types=[
          pltpu.VMEM((PER_W * C,), jnp.int32),
          pltpu.VMEM((PER_W * TN,), jnp.int32),
          pltpu.VMEM((E * C, W), jnp.int32),
          pltpu.VMEM((E * TN, W), jnp.int32),
          pltpu.VMEM((E * C, W), jnp.int32),
          pltpu.VMEM((E * TN, W), jnp.int32),
          pltpu.VMEM((PER_W, TN), jnp.float32),
          pltpu.SemaphoreType.DMA,
          pltpu.SemaphoreType.DMA,
      ],
      compiler_params=pltpu.CompilerParams(
          needs_layout_passes=False, use_tc_tiling_on_sc=False),
  )
  return f(ctx_idx, tn_idx, iw_packed, ow_packed)


def kernel(context_words, target_word, negative_samples, input_weight,
           output_weight):
  ctx_idx = context_words.astype(jnp.int32).reshape(-1)
  tn_idx = jnp.concatenate(
      [target_word.astype(jnp.int32)[:, None],
       negative_samples.astype(jnp.int32)], axis=1).reshape(-1)
  iw_packed = _pack_table(input_weight).reshape(V, W)
  ow_packed = _pack_table(output_weight).reshape(V, W)
  scores = _cbow_scores(ctx_idx, tn_idx, iw_packed, ow_packed)
  return scores[:, 0], scores[:, 1:]


# speedup vs baseline: 1.2037x; 1.0012x over previous
import jax
import jax.numpy as jnp
from jax import lax
from jax.experimental import pallas as pl
from jax.experimental.pallas import tpu as pltpu
from jax.experimental.pallas import tpu_sc as plsc

V = 1000000
D = 64
B = 16384
C = 20
NEG = 20
TN = NEG + 1
W = D // 2

NUM_WORKERS = 32
PER_W = B // NUM_WORKERS
E = 16
CHUNKS = PER_W // E
L = 16

PACK_RB = 200
PACK_NBLK = V // PACK_RB


def _pack_body(src, dst,
               x0, x1, w0, w1, si0, si1, so0, so1):
  wid = lax.axis_index("s") * 2 + lax.axis_index("c")
  base = PACK_NBLK // NUM_WORKERS
  cnt = base + jnp.where(wid < PACK_NBLK % NUM_WORKERS, 1, 0)
  start = wid * base + jnp.minimum(wid, PACK_NBLK % NUM_WORKERS)
  sets = ((x0, w0, si0, so0), (x1, w1, si1, so1))


  def fire_in(i, s):
    x, _, si, _ = s
    pltpu.async_copy(
        src.at[pl.ds((start + i) * PACK_RB, PACK_RB), :], x, si)

  def compute(s):
    x, w, _, _ = s

    @pl.loop(0, PACK_RB, unroll=8)
    def _rows(r):
      for q in range(2):
        a = x[r, pl.ds(q * L, L)]
        bb = x[r, pl.ds(W + q * L, L)]
        wv = plsc.bitcast(
            plsc.pack(a, bb, format=plsc.PackFormat.INTERLEAVED),
            jnp.int32)
        w[pl.ds(r * W + q * L, L)] = wv

  def proc(i, s):
    x, w, si, so = s
    pltpu.make_async_copy(
        src.at[pl.ds(0, PACK_RB), :], x, si).wait()

    @pl.when(i >= 2)
    def _dr():
      pltpu.make_async_copy(w, dst.at[pl.ds(0, PACK_RB * W)], so).wait()

    compute(s)
    pltpu.async_copy(w, dst.at[pl.ds((start + i) * PACK_RB * W,
                                     PACK_RB * W)], so)

  fire_in(0, sets[0])
  fire_in(1, sets[1])

  @pl.loop(0, cnt // 2)
  def _blocks(t):
    i = t * 2
    proc(i, sets[0])

    @pl.when(i + 2 < cnt)
    def _f0():
      fire_in(i + 2, sets[0])

    proc(i + 1, sets[1])

    @pl.when(i + 3 < cnt)
    def _f1():
      fire_in(i + 3, sets[1])

  @pl.when(cnt % 2 == 1)
  def _tail():
    proc(cnt - 1, sets[0])

  for s in sets:
    _, w, _, so = s
    pltpu.make_async_copy(w, dst.at[pl.ds(0, PACK_RB * W)], so).wait()


def _pack_table(table):
  mesh = plsc.VectorSubcoreMesh(core_axis_name="c", subcore_axis_name="s")
  f = pl.kernel(
      _pack_body,
      out_type=jax.ShapeDtypeStruct((V * W,), jnp.int32),
      mesh=mesh,
      scratch_types=[
          pltpu.VMEM((PACK_RB, D), jnp.float32),
          pltpu.VMEM((PACK_RB, D), jnp.float32),
          pltpu.VMEM((PACK_RB * W,), jnp.int32),
          pltpu.VMEM((PACK_RB * W,), jnp.int32),
          pltpu.SemaphoreType.DMA,
          pltpu.SemaphoreType.DMA,
          pltpu.SemaphoreType.DMA,
          pltpu.SemaphoreType.DMA,
      ],
      compiler_params=pltpu.CompilerParams(
          needs_layout_passes=False, use_tc_tiling_on_sc=True),
  )
  return f(table)


def _body(ctx_idx_hbm, tn_idx_hbm, iw_hbm, ow_hbm, out_hbm,
          ctx_idx_v, tn_idx_v, crows0, trows0, crows1, trows1,
          scores_v, sem0, sem1):
  wid = lax.axis_index("s") * 2 + lax.axis_index("c")
  wbase = wid * PER_W
  ev = lax.iota(jnp.int32, L)
  inv_c = jnp.float32(1.0 / C)
  evC = ev * C
  evTN = ev * TN
  himask = jnp.full((L,), -65536, jnp.int32)

  pltpu.sync_copy(ctx_idx_hbm.at[pl.ds(wbase * C, PER_W * C)], ctx_idx_v)
  pltpu.sync_copy(tn_idx_hbm.at[pl.ds(wbase * TN, PER_W * TN)], tn_idx_v)

  bufs = ((crows0, trows0, sem0), (crows1, trows1, sem1))

  def fire(c, crows, trows, sem):
    pltpu.async_copy(
        iw_hbm.at[ctx_idx_v.at[pl.ds(c * (E * C), E * C)]], crows, sem)
    pltpu.async_copy(
        ow_hbm.at[tn_idx_v.at[pl.ds(c * (E * TN), E * TN)]], trows, sem)

  def drain(crows, trows, sem):
    pltpu.make_async_copy(
        iw_hbm.at[ctx_idx_v.at[pl.ds(0, E * C)]], crows, sem).wait()
    pltpu.make_async_copy(
        ow_hbm.at[tn_idx_v.at[pl.ds(0, E * TN)]], trows, sem).wait()

  def unpack(word):
    f_lo = plsc.bitcast(word << 16, jnp.float32)
    f_hi = plsc.bitcast(word & himask, jnp.float32)
    return f_lo, f_hi

  def compute(c, crows, trows):
    cev = c * E + ev

    def word_scores(k):
      kcol = jnp.full((L,), k, jnp.int32)
      s_lo, s_hi = unpack(plsc.load_gather(crows, [evC, kcol]))
      for r in range(1, C):
        lo, hi = unpack(plsc.load_gather(crows, [evC + r, kcol]))
        s_lo = s_lo + lo
        s_hi = s_hi + hi
      a_lo = s_lo * inv_c
      a_hi = s_hi * inv_c
      out = []
      for j in range(TN):
        lo, hi = unpack(plsc.load_gather(trows, [evTN + j, kcol]))
        out.append(a_lo * lo + a_hi * hi)
      return out

    sc0 = word_scores(0)
    for j in range(TN):
      plsc.store_scatter(
          scores_v, [cev, jnp.full((L,), j, jnp.int32)], sc0[j])

    @pl.loop(1, W)
    def _words(k):
      sck = word_scores(k)
      for j in range(TN):
        plsc.addupdate_scatter(
            scores_v, [cev, jnp.full((L,), j, jnp.int32)], sck[j])

  fire(0, *bufs[0])

  @pl.loop(0, CHUNKS, step=2)
  def _chunks(c):
    fire(c + 1, *bufs[1])
    drain(*bufs[0])
    compute(c, bufs[0][0], bufs[0][1])

    @pl.when(c + 2 < CHUNKS)
    def _prefetch():
      fire(c + 2, *bufs[0])

    drain(*bufs[1])
    compute(c + 1, bufs[1][0], bufs[1][1])

  pltpu.sync_copy(scores_v, out_hbm.at[pl.ds(wbase, PER_W)])


def _cbow_scores(ctx_idx, tn_idx, iw_packed, ow_packed):
  mesh = plsc.VectorSubcoreMesh(core_axis_name="c", subcore_axis_name="s")
  f = pl.kernel(
      _body,
      out_type=jax.ShapeDtypeStruct((B, TN), jnp.float32),
      mesh=mesh,
      scratch_types=[
          pltpu.VMEM((PER_W * C,), jnp.int32),
          pltpu.VMEM((PER_W * TN,), jnp.int32),
          pltpu.VMEM((E * C, W), jnp.int32),
          pltpu.VMEM((E * TN, W), jnp.int32),
          pltpu.VMEM((E * C, W), jnp.int32),
          pltpu.VMEM((E * TN, W), jnp.int32),
          pltpu.VMEM((PER_W, TN), jnp.float32),
          pltpu.SemaphoreType.DMA,
          pltpu.SemaphoreType.DMA,
      ],
      compiler_params=pltpu.CompilerParams(
          needs_layout_passes=False, use_tc_tiling_on_sc=False),
  )
  return f(ctx_idx, tn_idx, iw_packed, ow_packed)


def kernel(context_words, target_word, negative_samples, input_weight,
           output_weight):
  ctx_idx = context_words.astype(jnp.int32).reshape(-1)
  tn_idx = jnp.concatenate(
      [target_word.astype(jnp.int32)[:, None],
       negative_samples.astype(jnp.int32)], axis=1).reshape(-1)
  iw_packed = _pack_table(input_weight).reshape(V, W)
  ow_packed = _pack_table(output_weight).reshape(V, W)
  scores = _cbow_scores(ctx_idx, tn_idx, iw_packed, ow_packed)
  return scores[:, 0], scores[:, 1:]
